# baseline (speedup 1.0000x reference)
import functools

import jax
import jax.numpy as jnp
from jax import lax
from jax.experimental import pallas as pl
from jax.experimental.pallas import tpu as pltpu
from jax.experimental.pallas import tpu_sc as plsc

_B, _N, _CIN, _COUT = 8, 131072, 20, 13
_NC, _NS = 2, 16
_NW = _NC * _NS
_COLS_W = _N // _NW
_W = 128
_CHUNKS = _COLS_W // _W
_POS = _B * _W
_GROUPS = _POS // 16

_TGT_SRCS = [
    [], [1], [0], [], [], [8], [7], [6, 12], [4], [5], [9], [],
    [2, 3, 10, 11, 13, 14, 15, 16, 17, 18, 19],
]


def _sc_body(in_hbm, out_hbm, in_v, out_v):
    wid = lax.axis_index("s") * _NC + lax.axis_index("c")
    col0 = wid * _COLS_W

    def chunk_body(c, carry):
        off = col0 + c * _W
        pltpu.sync_copy(in_hbm.at[:, :, pl.ds(off, _W)], in_v)

        @plsc.parallel_loop(0, _GROUPS, unroll=4)
        def group_body(g):
            r = g >> 3
            k = (g & 7) * 16
            v = [in_v[s, r, pl.ds(k, 16)] for s in range(_CIN)]
            for t, srcs in enumerate(_TGT_SRCS):
                if not srcs:
                    out_v[t, r, pl.ds(k, 16)] = jnp.full(
                        (16,), -jnp.inf, dtype=jnp.float32)
                else:
                    acc = [v[s] for s in srcs]
                    while len(acc) > 1:
                        acc = [jnp.maximum(a, b)
                               for a, b in zip(acc[::2], acc[1::2])] + (
                            [acc[-1]] if len(acc) % 2 else [])
                    out_v[t, r, pl.ds(k, 16)] = acc[0]

        pltpu.sync_copy(out_v, out_hbm.at[:, :, pl.ds(off, _W)])
        return carry

    lax.fori_loop(0, _CHUNKS, chunk_body, 0)


@functools.partial(jax.jit, static_argnums=())
def kernel(logits):
    xt = jnp.transpose(logits, (2, 0, 1))
    run = pl.kernel(
        _sc_body,
        out_type=jax.ShapeDtypeStruct((_COUT, _B, _N), jnp.float32),
        mesh=plsc.VectorSubcoreMesh(core_axis_name="c", subcore_axis_name="s"),
        compiler_params=pltpu.CompilerParams(
            needs_layout_passes=False, use_tc_tiling_on_sc=False),
        scratch_types=[
            pltpu.VMEM((_CIN, _B, _W), jnp.float32),
            pltpu.VMEM((_COUT, _B, _W), jnp.float32),
        ],
    )
    out = run(xt)
    return jnp.transpose(out, (1, 2, 0))


# speedup vs baseline: 4.4646x; 4.4646x over previous
import functools

import jax
import jax.numpy as jnp
from jax.experimental import pallas as pl

_B, _N, _CIN, _COUT = 8, 131072, 20, 13
_BLK = 4096

_TGT_SRCS = [
    [], [1], [0], [], [], [8], [7], [6, 12], [4], [5], [9], [],
    [2, 3, 10, 11, 13, 14, 15, 16, 17, 18, 19],
]


def _tc_body(x_ref, o_ref):
    for t, srcs in enumerate(_TGT_SRCS):
        if not srcs:
            o_ref[t] = jnp.full((_B, _BLK), -jnp.inf, dtype=jnp.float32)
        else:
            acc = [x_ref[s] for s in srcs]
            while len(acc) > 1:
                acc = [jnp.maximum(a, b) for a, b in zip(acc[::2], acc[1::2])] + (
                    [acc[-1]] if len(acc) % 2 else [])
            o_ref[t] = acc[0]


@functools.partial(jax.jit, static_argnums=())
def kernel(logits):
    xt = jnp.transpose(logits, (2, 0, 1))
    out = pl.pallas_call(
        _tc_body,
        grid=(_N // _BLK,),
        in_specs=[pl.BlockSpec((_CIN, _B, _BLK), lambda i: (0, 0, i))],
        out_specs=pl.BlockSpec((_COUT, _B, _BLK), lambda i: (0, 0, i)),
        out_shape=jax.ShapeDtypeStruct((_COUT, _B, _N), jnp.float32),
    )(xt)
    return jnp.transpose(out, (1, 2, 0))


# speedup vs baseline: 4.8378x; 1.0836x over previous
import functools

import jax
import jax.numpy as jnp
from jax.experimental import pallas as pl

_B, _N, _CIN, _COUT = 8, 131072, 20, 13
_BLK = 8192

_TGT_SRCS = [
    [], [1], [0], [], [], [8], [7], [6, 12], [4], [5], [9], [],
    [2, 3, 10, 11, 13, 14, 15, 16, 17, 18, 19],
]


def _tc_body(x_ref, o_ref):
    for t, srcs in enumerate(_TGT_SRCS):
        if not srcs:
            o_ref[t] = jnp.full((_B, _BLK), -jnp.inf, dtype=jnp.float32)
        else:
            acc = [x_ref[s] for s in srcs]
            while len(acc) > 1:
                acc = [jnp.maximum(a, b) for a, b in zip(acc[::2], acc[1::2])] + (
                    [acc[-1]] if len(acc) % 2 else [])
            o_ref[t] = acc[0]


@functools.partial(jax.jit, static_argnums=())
def kernel(logits):
    xt = jnp.transpose(logits, (2, 0, 1))
    out = pl.pallas_call(
        _tc_body,
        grid=(_N // _BLK,),
        in_specs=[pl.BlockSpec((_CIN, _B, _BLK), lambda i: (0, 0, i))],
        out_specs=pl.BlockSpec((_COUT, _B, _BLK), lambda i: (0, 0, i)),
        out_shape=jax.ShapeDtypeStruct((_COUT, _B, _N), jnp.float32),
    )(xt)
    return jnp.transpose(out, (1, 2, 0))


# speedup vs baseline: 4.9058x; 1.0141x over previous
import functools

import jax
import jax.numpy as jnp
from jax.experimental import pallas as pl

_B, _N, _CIN, _COUT = 8, 131072, 20, 13
_BLK = 16384

_TGT_SRCS = [
    [], [1], [0], [], [], [8], [7], [6, 12], [4], [5], [9], [],
    [2, 3, 10, 11, 13, 14, 15, 16, 17, 18, 19],
]


def _tc_body(x_ref, o_ref):
    for t, srcs in enumerate(_TGT_SRCS):
        if not srcs:
            o_ref[t] = jnp.full((_B, _BLK), -jnp.inf, dtype=jnp.float32)
        else:
            acc = [x_ref[s] for s in srcs]
            while len(acc) > 1:
                acc = [jnp.maximum(a, b) for a, b in zip(acc[::2], acc[1::2])] + (
                    [acc[-1]] if len(acc) % 2 else [])
            o_ref[t] = acc[0]


@functools.partial(jax.jit, static_argnums=())
def kernel(logits):
    xt = jnp.transpose(logits, (2, 0, 1))
    out = pl.pallas_call(
        _tc_body,
        grid=(_N // _BLK,),
        in_specs=[pl.BlockSpec((_CIN, _B, _BLK), lambda i: (0, 0, i))],
        out_specs=pl.BlockSpec((_COUT, _B, _BLK), lambda i: (0, 0, i)),
        out_shape=jax.ShapeDtypeStruct((_COUT, _B, _N), jnp.float32),
    )(xt)
    return jnp.transpose(out, (1, 2, 0))
